# initial kernel scaffold (unmeasured)
import jax
import jax.numpy as jnp
from jax import lax
from jax.experimental import pallas as pl
from jax.experimental.pallas import tpu as pltpu

B = 4
S = 1024
S_HALF = 512
K = 2048
N = 4096
N_ROWS = B * S
HALF = B * S_HALF
CHUNK = 256
N_CHUNKS = HALF // CHUNK


def kernel(O, Wo):
    o_flat = O.reshape(N_ROWS, K)

    def body(o_hbm, wo_hbm, out_hbm, recv_hbm,
             wo_vmem, o_slots, send_slots, rv_vmem, fin_slots,
             wo_sem, o_sems, send_sems, recv_sems, rv_sem, fin_sems):
        my_x = lax.axis_index("x")
        my_y = lax.axis_index("y")
        my_z = lax.axis_index("z")
        peer = (1 - my_x, my_y, my_z)

        wo_cp = pltpu.make_async_copy(wo_hbm, wo_vmem, wo_sem)
        wo_cp.start()

        barrier = pltpu.get_barrier_semaphore()
        pl.semaphore_signal(barrier, inc=1, device_id=peer,
                            device_id_type=pl.DeviceIdType.MESH)
        pl.semaphore_wait(barrier, 1)

        def o_row(c, half):
            return (c // 2) * S + half * S_HALF + (c % 2) * CHUNK

        halves = [1 - my_x] * N_CHUNKS + [my_x] * N_CHUNKS
        rows = [o_row(k % N_CHUNKS, halves[k]) for k in range(2 * N_CHUNKS)]

        def start_o_load(k):
            cp = pltpu.make_async_copy(
                o_hbm.at[pl.ds(rows[k], CHUNK)],
                o_slots.at[k % 2],
                o_sems.at[k % 2],
            )
            cp.start()
            return cp

        o_cps = {0: start_o_load(0)}
        wo_cp.wait()

        rdmas = []
        for c in range(N_CHUNKS):
            k = c
            if k + 1 < 2 * N_CHUNKS:
                o_cps[k + 1] = start_o_load(k + 1)
            o_cps[k].wait()
            s = c % 2
            if c >= 2:
                rdmas[c - 2].wait_send()
            send_slots[s] = jnp.dot(
                o_slots[s], wo_vmem[...], preferred_element_type=jnp.float32
            )
            rdma = pltpu.make_async_remote_copy(
                src_ref=send_slots.at[s],
                dst_ref=recv_hbm.at[pl.ds(c * CHUNK, CHUNK)],
                send_sem=send_sems.at[s],
                recv_sem=recv_sems.at[c],
                device_id=peer,
                device_id_type=pl.DeviceIdType.MESH,
            )
            rdma.start()
            rdmas.append(rdma)

        fin_cps = []
        for c in range(N_CHUNKS):
            k = N_CHUNKS + c
            if k + 1 < 2 * N_CHUNKS:
                o_cps[k + 1] = start_o_load(k + 1)
            o_cps[k].wait()
            s = c % 2
            if c >= 2:
                fin_cps[c - 2].wait()
            part = jnp.dot(
                o_slots[k % 2], wo_vmem[...],
                preferred_element_type=jnp.float32,
            )
            rdmas[c].wait_recv()
            rv_cp = pltpu.make_async_copy(
                recv_hbm.at[pl.ds(c * CHUNK, CHUNK)], rv_vmem, rv_sem)
            rv_cp.start()
            rv_cp.wait()
            fin_slots[s] = part + rv_vmem[...]
            cp = pltpu.make_async_copy(
                fin_slots.at[s],
                out_hbm.at[pl.ds(c * CHUNK, CHUNK)],
                fin_sems.at[s],
            )
            cp.start()
            fin_cps.append(cp)

        fin_cps[N_CHUNKS - 2].wait()
        fin_cps[N_CHUNKS - 1].wait()
        rdmas[N_CHUNKS - 2].wait_send()
        rdmas[N_CHUNKS - 1].wait_send()

    out, _recv = pl.pallas_call(
        body,
        out_shape=[
            jax.ShapeDtypeStruct((HALF, N), jnp.float32),
            jax.ShapeDtypeStruct((HALF, N), jnp.float32),
        ],
        in_specs=[
            pl.BlockSpec(memory_space=pltpu.MemorySpace.HBM),
            pl.BlockSpec(memory_space=pltpu.MemorySpace.HBM),
        ],
        out_specs=[
            pl.BlockSpec(memory_space=pltpu.MemorySpace.HBM),
            pl.BlockSpec(memory_space=pltpu.MemorySpace.HBM),
        ],
        scratch_shapes=[
            pltpu.VMEM((K, N), jnp.float32),
            pltpu.VMEM((2, CHUNK, K), jnp.float32),
            pltpu.VMEM((2, CHUNK, N), jnp.float32),
            pltpu.VMEM((CHUNK, N), jnp.float32),
            pltpu.VMEM((2, CHUNK, N), jnp.float32),
            pltpu.SemaphoreType.DMA,
            pltpu.SemaphoreType.DMA((2,)),
            pltpu.SemaphoreType.DMA((2,)),
            pltpu.SemaphoreType.DMA((N_CHUNKS,)),
            pltpu.SemaphoreType.DMA,
            pltpu.SemaphoreType.DMA((2,)),
        ],
        compiler_params=pltpu.CompilerParams(collective_id=0),
    )(o_flat, Wo)
    return out.reshape(B, S_HALF, N)


# baseline (device time: 456274 ns/iter reference)
import jax
import jax.numpy as jnp
from jax import lax
from jax.experimental import pallas as pl
from jax.experimental.pallas import tpu as pltpu

B = 4
S = 1024
S_HALF = 512
K = 2048
N = 4096
N_ROWS = B * S
HALF = B * S_HALF
CHUNK = 256
N_CHUNKS = HALF // CHUNK


def kernel(O, Wo):
    o_flat = O.reshape(N_ROWS, K)

    def body(o_hbm, wo_hbm, out_hbm, recv_hbm,
             wo_vmem, o_slots, send_slots, rv_vmem, fin_slots,
             wo_sem, o_sems, send_sems, recv_sems, rv_sem, fin_sems):
        my_x = lax.axis_index("x")
        my_y = lax.axis_index("y")
        my_z = lax.axis_index("z")
        peer = (1 - my_x, my_y, my_z)

        wo_cp = pltpu.make_async_copy(wo_hbm, wo_vmem, wo_sem)
        wo_cp.start()

        barrier = pltpu.get_barrier_semaphore()
        pl.semaphore_signal(barrier, inc=1, device_id=peer,
                            device_id_type=pl.DeviceIdType.MESH)
        pl.semaphore_wait(barrier, 1)

        def o_row(c, half):
            return (c // 2) * S + half * S_HALF + (c % 2) * CHUNK

        halves = [1 - my_x] * N_CHUNKS + [my_x] * N_CHUNKS
        rows = [o_row(k % N_CHUNKS, halves[k]) for k in range(2 * N_CHUNKS)]

        def start_o_load(k):
            cp = pltpu.make_async_copy(
                o_hbm.at[pl.ds(rows[k], CHUNK)],
                o_slots.at[k % 2],
                o_sems.at[k % 2],
            )
            cp.start()
            return cp

        o_cps = {0: start_o_load(0)}
        wo_cp.wait()

        rdmas = []
        for c in range(N_CHUNKS):
            k = c
            if k + 1 < 2 * N_CHUNKS:
                o_cps[k + 1] = start_o_load(k + 1)
            o_cps[k].wait()
            s = c % 2
            if c >= 2:
                rdmas[c - 2].wait_send()
            send_slots[s] = jnp.dot(
                o_slots[s], wo_vmem[...], preferred_element_type=jnp.float32
            )
            rdma = pltpu.make_async_remote_copy(
                src_ref=send_slots.at[s],
                dst_ref=recv_hbm.at[pl.ds(c * CHUNK, CHUNK)],
                send_sem=send_sems.at[s],
                recv_sem=recv_sems.at[c],
                device_id=peer,
                device_id_type=pl.DeviceIdType.MESH,
            )
            rdma.start()
            rdmas.append(rdma)

        fin_cps = []
        for c in range(N_CHUNKS):
            k = N_CHUNKS + c
            if k + 1 < 2 * N_CHUNKS:
                o_cps[k + 1] = start_o_load(k + 1)
            o_cps[k].wait()
            s = c % 2
            if c >= 2:
                fin_cps[c - 2].wait()
            fin_slots[s] = jnp.dot(
                o_slots[k % 2], wo_vmem[...],
                preferred_element_type=jnp.float32,
            )
            rdmas[c].wait_recv()
            rv_cp = pltpu.make_async_copy(
                recv_hbm.at[pl.ds(c * CHUNK, CHUNK)], rv_vmem, rv_sem)
            rv_cp.start()
            rv_cp.wait()
            fin_slots[s] = fin_slots[s] + rv_vmem[...]
            cp = pltpu.make_async_copy(
                fin_slots.at[s],
                out_hbm.at[pl.ds(c * CHUNK, CHUNK)],
                fin_sems.at[s],
            )
            cp.start()
            fin_cps.append(cp)

        fin_cps[N_CHUNKS - 2].wait()
        fin_cps[N_CHUNKS - 1].wait()
        rdmas[N_CHUNKS - 2].wait_send()
        rdmas[N_CHUNKS - 1].wait_send()

    out, _recv = pl.pallas_call(
        body,
        out_shape=[
            jax.ShapeDtypeStruct((HALF, N), jnp.float32),
            jax.ShapeDtypeStruct((HALF, N), jnp.float32),
        ],
        in_specs=[
            pl.BlockSpec(memory_space=pltpu.MemorySpace.HBM),
            pl.BlockSpec(memory_space=pltpu.MemorySpace.HBM),
        ],
        out_specs=[
            pl.BlockSpec(memory_space=pltpu.MemorySpace.HBM),
            pl.BlockSpec(memory_space=pltpu.MemorySpace.HBM),
        ],
        scratch_shapes=[
            pltpu.VMEM((K, N), jnp.float32),
            pltpu.VMEM((2, CHUNK, K), jnp.float32),
            pltpu.VMEM((2, CHUNK, N), jnp.float32),
            pltpu.VMEM((CHUNK, N), jnp.float32),
            pltpu.VMEM((2, CHUNK, N), jnp.float32),
            pltpu.SemaphoreType.DMA,
            pltpu.SemaphoreType.DMA((2,)),
            pltpu.SemaphoreType.DMA((2,)),
            pltpu.SemaphoreType.DMA((N_CHUNKS,)),
            pltpu.SemaphoreType.DMA,
            pltpu.SemaphoreType.DMA((2,)),
        ],
        compiler_params=pltpu.CompilerParams(
            collective_id=0,
            vmem_limit_bytes=63 * 1024 * 1024,
        ),
    )(o_flat, Wo)
    return out.reshape(B, S_HALF, N)


# device time: 455316 ns/iter; 1.0021x vs baseline; 1.0021x over previous
import jax
import jax.numpy as jnp
from jax import lax
from jax.experimental import pallas as pl
from jax.experimental.pallas import tpu as pltpu

B = 4
S = 1024
S_HALF = 512
K = 2048
N = 4096
N_ROWS = B * S
HALF = B * S_HALF
CHUNK = 256
MESH = pl.DeviceIdType.MESH


def kernel(O, Wo):
    o_flat = O.reshape(N_ROWS, K)

    def body(o_hbm, wo_hbm, out_hbm,
             wo_vmem, o_slots, xsend, xrecv, red,
             wo_sem, o_sems, xsend_sems, xrecv_sems,
             zsend_sems, ysend_sems, fwd_send_sems,
             zrecv_sems, yrecv_sems, out_sems):
        my_x = lax.axis_index("x")
        my_y = lax.axis_index("y")
        my_z = lax.axis_index("z")
        xpeer = (1 - my_x, my_y, my_z)
        ypeer = (my_x, 1 - my_y, my_z)
        zpeer = (my_x, my_y, 1 - my_z)
        q = 2 * my_y + my_z
        q_y = 2 * (1 - my_y) + my_z
        q_z = 2 * my_y + (1 - my_z)
        q_g = 2 * (1 - my_y) + (1 - my_z)

        wo_cp = pltpu.make_async_copy(wo_hbm, wo_vmem, wo_sem)
        wo_cp.start()

        barrier = pltpu.get_barrier_semaphore()
        for nbr in (xpeer, ypeer, zpeer):
            pl.semaphore_signal(barrier, inc=1, device_id=nbr,
                                device_id_type=MESH)
        pl.semaphore_wait(barrier, 3)

        rows = [q * S + (1 - my_x) * S_HALF + 0,
                q * S + (1 - my_x) * S_HALF + CHUNK,
                q * S + my_x * S_HALF + 0,
                q * S + my_x * S_HALF + CHUNK]

        def start_o_load(k):
            cp = pltpu.make_async_copy(
                o_hbm.at[pl.ds(rows[k], CHUNK)],
                o_slots.at[k % 2],
                o_sems.at[k % 2],
            )
            cp.start()
            return cp

        o_cps = {0: start_o_load(0)}
        wo_cp.wait()

        x_rdmas = []
        for c in range(2):
            o_cps[c + 1] = start_o_load(c + 1)
            o_cps[c].wait()
            xsend[c] = jnp.dot(o_slots[c % 2], wo_vmem[...],
                               preferred_element_type=jnp.float32)
            rdma = pltpu.make_async_remote_copy(
                src_ref=xsend.at[c],
                dst_ref=xrecv.at[c],
                send_sem=xsend_sems.at[c],
                recv_sem=xrecv_sems.at[c],
                device_id=xpeer,
                device_id_type=MESH,
            )
            rdma.start()
            x_rdmas.append(rdma)

        for c in range(2):
            k = 2 + c
            if k + 1 < 4:
                o_cps[k + 1] = start_o_load(k + 1)
            o_cps[k].wait()
            red[pl.ds(c * CHUNK, CHUNK)] = jnp.dot(
                o_slots[k % 2], wo_vmem[...],
                preferred_element_type=jnp.float32)

        def gather_send(src_ref, row0, dev, send_sem, recv_sem):
            rdma = pltpu.make_async_remote_copy(
                src_ref=src_ref,
                dst_ref=out_hbm.at[pl.ds(row0, CHUNK)],
                send_sem=send_sem,
                recv_sem=recv_sem,
                device_id=dev,
                device_id_type=MESH,
            )
            rdma.start()
            return rdma

        gather_rdmas = []
        out_cps = []
        for c in range(2):
            x_rdmas[c].wait_recv()
            red[pl.ds(c * CHUNK, CHUNK)] = (
                red[pl.ds(c * CHUNK, CHUNK)] + xrecv[c])
            gather_rdmas.append(gather_send(
                red.at[pl.ds(c * CHUNK, CHUNK)], q * S_HALF + c * CHUNK,
                zpeer, zsend_sems.at[c], zrecv_sems.at[c]))
            gather_rdmas.append(gather_send(
                red.at[pl.ds(c * CHUNK, CHUNK)], q * S_HALF + c * CHUNK,
                ypeer, ysend_sems.at[c], yrecv_sems.at[c]))
            cp = pltpu.make_async_copy(
                red.at[pl.ds(c * CHUNK, CHUNK)],
                out_hbm.at[pl.ds(q * S_HALF + c * CHUNK, CHUNK)],
                out_sems.at[c],
            )
            cp.start()
            out_cps.append(cp)

        def recv_desc(row0, sem):
            return pltpu.make_async_remote_copy(
                src_ref=red.at[pl.ds(0, CHUNK)],
                dst_ref=out_hbm.at[pl.ds(row0, CHUNK)],
                send_sem=fwd_send_sems.at[0],
                recv_sem=sem,
                device_id=xpeer,
                device_id_type=MESH,
            )

        zr = [recv_desc(q_z * S_HALF + c * CHUNK, zrecv_sems.at[c])
              for c in range(2)]
        yr = [recv_desc(q_y * S_HALF + c * CHUNK, yrecv_sems.at[c])
              for c in range(2)]
        zrf = recv_desc(q_g * S_HALF + CHUNK, zrecv_sems.at[2])
        yrf = recv_desc(q_g * S_HALF, yrecv_sems.at[2])

        zr[0].wait_recv()
        yf = gather_send(out_hbm.at[pl.ds(q_z * S_HALF, CHUNK)],
                         q_z * S_HALF, ypeer,
                         fwd_send_sems.at[0], yrecv_sems.at[2])
        yr[1].wait_recv()
        zf = gather_send(out_hbm.at[pl.ds(q_y * S_HALF + CHUNK, CHUNK)],
                         q_y * S_HALF + CHUNK, zpeer,
                         fwd_send_sems.at[1], zrecv_sems.at[2])

        zr[1].wait_recv()
        yr[0].wait_recv()
        yrf.wait_recv()
        zrf.wait_recv()

        for cp in out_cps:
            cp.wait()
        for rdma in x_rdmas:
            rdma.wait_send()
        for rdma in gather_rdmas:
            rdma.wait_send()
        yf.wait_send()
        zf.wait_send()

    out = pl.pallas_call(
        body,
        out_shape=jax.ShapeDtypeStruct((HALF, N), jnp.float32),
        in_specs=[
            pl.BlockSpec(memory_space=pltpu.MemorySpace.HBM),
            pl.BlockSpec(memory_space=pltpu.MemorySpace.HBM),
        ],
        out_specs=pl.BlockSpec(memory_space=pltpu.MemorySpace.HBM),
        scratch_shapes=[
            pltpu.VMEM((K, N), jnp.float32),
            pltpu.VMEM((2, CHUNK, K), jnp.float32),
            pltpu.VMEM((2, CHUNK, N), jnp.float32),
            pltpu.VMEM((2, CHUNK, N), jnp.float32),
            pltpu.VMEM((S_HALF, N), jnp.float32),
            pltpu.SemaphoreType.DMA,
            pltpu.SemaphoreType.DMA((2,)),
            pltpu.SemaphoreType.DMA((2,)),
            pltpu.SemaphoreType.DMA((2,)),
            pltpu.SemaphoreType.DMA((2,)),
            pltpu.SemaphoreType.DMA((2,)),
            pltpu.SemaphoreType.DMA((2,)),
            pltpu.SemaphoreType.DMA((3,)),
            pltpu.SemaphoreType.DMA((3,)),
            pltpu.SemaphoreType.DMA((2,)),
        ],
        compiler_params=pltpu.CompilerParams(
            collective_id=0,
            vmem_limit_bytes=63 * 1024 * 1024,
        ),
    )(o_flat, Wo)
    return out.reshape(B, S_HALF, N)


# device time: 236057 ns/iter; 1.9329x vs baseline; 1.9288x over previous
import jax
import jax.numpy as jnp
from jax import lax
from jax.experimental import pallas as pl
from jax.experimental.pallas import tpu as pltpu

B = 4
S = 1024
S_HALF = 512
H = 16
D = 128
K = H * D
N = 4096
HALF = B * S_HALF
CHUNK = 256
MESH = pl.DeviceIdType.MESH


def kernel(O, Wo):

    def body(o_hbm, wo_hbm, out_hbm, xrecv_hbm,
             wo_vmem, o_slots, xsend, rv_vmem, red,
             wo_sem, o_sems, rv_sem, xsend_sems, xrecv_sems,
             zsend_sems, ysend_sems, fwd_send_sems,
             zrecv_sems, yrecv_sems, out_sems):
        my_x = lax.axis_index("x")
        my_y = lax.axis_index("y")
        my_z = lax.axis_index("z")
        xpeer = (1 - my_x, my_y, my_z)
        ypeer = (my_x, 1 - my_y, my_z)
        zpeer = (my_x, my_y, 1 - my_z)
        q = 2 * my_y + my_z
        q_y = 2 * (1 - my_y) + my_z
        q_z = 2 * my_y + (1 - my_z)
        q_g = 2 * (1 - my_y) + (1 - my_z)

        wo_cp = pltpu.make_async_copy(wo_hbm, wo_vmem, wo_sem)
        wo_cp.start()

        barrier = pltpu.get_barrier_semaphore()
        for nbr in (xpeer, ypeer, zpeer):
            pl.semaphore_signal(barrier, inc=1, device_id=nbr,
                                device_id_type=MESH)
        pl.semaphore_wait(barrier, 3)

        s0s = [(1 - my_x) * S_HALF + 0,
               (1 - my_x) * S_HALF + CHUNK,
               my_x * S_HALF + 0,
               my_x * S_HALF + CHUNK]

        def start_o_load(k):
            cps = []
            for h in range(H):
                cp = pltpu.make_async_copy(
                    o_hbm.at[q, pl.ds(s0s[k], CHUNK), h],
                    o_slots.at[k % 2, h],
                    o_sems.at[k % 2, h],
                )
                cp.start()
                cps.append(cp)
            return cps

        def head_matmul(s):
            acc = jnp.dot(o_slots[s, 0], wo_vmem[pl.ds(0, D)],
                          preferred_element_type=jnp.float32)
            for h in range(1, H):
                acc = acc + jnp.dot(o_slots[s, h], wo_vmem[pl.ds(h * D, D)],
                                    preferred_element_type=jnp.float32)
            return acc

        o_cps = {0: start_o_load(0)}
        wo_cp.wait()

        x_rdmas = []
        for c in range(2):
            o_cps[c + 1] = start_o_load(c + 1)
            for cp in o_cps[c]:
                cp.wait()
            xsend[c] = head_matmul(c % 2)
            rdma = pltpu.make_async_remote_copy(
                src_ref=xsend.at[c],
                dst_ref=xrecv_hbm.at[pl.ds(c * CHUNK, CHUNK)],
                send_sem=xsend_sems.at[c],
                recv_sem=xrecv_sems.at[c],
                device_id=xpeer,
                device_id_type=MESH,
            )
            rdma.start()
            x_rdmas.append(rdma)

        for c in range(2):
            k = 2 + c
            if k + 1 < 4:
                o_cps[k + 1] = start_o_load(k + 1)
            for cp in o_cps[k]:
                cp.wait()
            red[pl.ds(c * CHUNK, CHUNK)] = head_matmul(k % 2)

        def gather_send(src_ref, row0, dev, send_sem, recv_sem):
            rdma = pltpu.make_async_remote_copy(
                src_ref=src_ref,
                dst_ref=out_hbm.at[pl.ds(row0, CHUNK)],
                send_sem=send_sem,
                recv_sem=recv_sem,
                device_id=dev,
                device_id_type=MESH,
            )
            rdma.start()
            return rdma

        gather_rdmas = []
        out_cps = []
        for c in range(2):
            x_rdmas[c].wait_recv()
            rv_cp = pltpu.make_async_copy(
                xrecv_hbm.at[pl.ds(c * CHUNK, CHUNK)], rv_vmem, rv_sem)
            rv_cp.start()
            rv_cp.wait()
            red[pl.ds(c * CHUNK, CHUNK)] = (
                red[pl.ds(c * CHUNK, CHUNK)] + rv_vmem[...])
            gather_rdmas.append(gather_send(
                red.at[pl.ds(c * CHUNK, CHUNK)], q * S_HALF + c * CHUNK,
                zpeer, zsend_sems.at[c], zrecv_sems.at[c]))
            gather_rdmas.append(gather_send(
                red.at[pl.ds(c * CHUNK, CHUNK)], q * S_HALF + c * CHUNK,
                ypeer, ysend_sems.at[c], yrecv_sems.at[c]))
            cp = pltpu.make_async_copy(
                red.at[pl.ds(c * CHUNK, CHUNK)],
                out_hbm.at[pl.ds(q * S_HALF + c * CHUNK, CHUNK)],
                out_sems.at[c],
            )
            cp.start()
            out_cps.append(cp)

        def recv_desc(row0, sem):
            return pltpu.make_async_remote_copy(
                src_ref=red.at[pl.ds(0, CHUNK)],
                dst_ref=out_hbm.at[pl.ds(row0, CHUNK)],
                send_sem=fwd_send_sems.at[0],
                recv_sem=sem,
                device_id=xpeer,
                device_id_type=MESH,
            )

        zr = [recv_desc(q_z * S_HALF + c * CHUNK, zrecv_sems.at[c])
              for c in range(2)]
        yr = [recv_desc(q_y * S_HALF + c * CHUNK, yrecv_sems.at[c])
              for c in range(2)]
        zrf = recv_desc(q_g * S_HALF + CHUNK, zrecv_sems.at[2])
        yrf = recv_desc(q_g * S_HALF, yrecv_sems.at[2])

        zr[0].wait_recv()
        yf = gather_send(out_hbm.at[pl.ds(q_z * S_HALF, CHUNK)],
                         q_z * S_HALF, ypeer,
                         fwd_send_sems.at[0], yrecv_sems.at[2])
        yr[1].wait_recv()
        zf = gather_send(out_hbm.at[pl.ds(q_y * S_HALF + CHUNK, CHUNK)],
                         q_y * S_HALF + CHUNK, zpeer,
                         fwd_send_sems.at[1], zrecv_sems.at[2])

        zr[1].wait_recv()
        yr[0].wait_recv()
        yrf.wait_recv()
        zrf.wait_recv()

        for cp in out_cps:
            cp.wait()
        for rdma in x_rdmas:
            rdma.wait_send()
        for rdma in gather_rdmas:
            rdma.wait_send()
        yf.wait_send()
        zf.wait_send()

    out, _xrecv = pl.pallas_call(
        body,
        out_shape=[
            jax.ShapeDtypeStruct((HALF, N), jnp.float32),
            jax.ShapeDtypeStruct((2 * CHUNK, N), jnp.float32),
        ],
        in_specs=[
            pl.BlockSpec(memory_space=pltpu.MemorySpace.HBM),
            pl.BlockSpec(memory_space=pltpu.MemorySpace.HBM),
        ],
        out_specs=[
            pl.BlockSpec(memory_space=pltpu.MemorySpace.HBM),
            pl.BlockSpec(memory_space=pltpu.MemorySpace.HBM),
        ],
        scratch_shapes=[
            pltpu.VMEM((K, N), jnp.float32),
            pltpu.VMEM((2, H, CHUNK, D), jnp.float32),
            pltpu.VMEM((2, CHUNK, N), jnp.float32),
            pltpu.VMEM((CHUNK, N), jnp.float32),
            pltpu.VMEM((S_HALF, N), jnp.float32),
            pltpu.SemaphoreType.DMA,
            pltpu.SemaphoreType.DMA((2, H)),
            pltpu.SemaphoreType.DMA,
            pltpu.SemaphoreType.DMA((2,)),
            pltpu.SemaphoreType.DMA((2,)),
            pltpu.SemaphoreType.DMA((2,)),
            pltpu.SemaphoreType.DMA((2,)),
            pltpu.SemaphoreType.DMA((2,)),
            pltpu.SemaphoreType.DMA((3,)),
            pltpu.SemaphoreType.DMA((3,)),
            pltpu.SemaphoreType.DMA((2,)),
        ],
        compiler_params=pltpu.CompilerParams(
            collective_id=0,
            vmem_limit_bytes=63 * 1024 * 1024,
        ),
    )(O, Wo)
    del _xrecv
    return out.reshape(B, S_HALF, N)
